# baseline (speedup 1.0000x reference)
import functools

import jax
import jax.numpy as jnp
import numpy as np
from jax import lax
from jax.experimental import pallas as pl
from jax.experimental.pallas import tpu as pltpu
from jax.experimental.pallas import tpu_sc as plsc

B = 4096
L = 200
V = 40961
D = 128
HALF = 100
NW = 32
SPW = B // NW
NCHUNK = SPW * 4
LANES = 16


def _bag_sum_body(idx_hbm, emb_hbm, out_hbm,
                  idx_v, rb0, rb1, out_v, sem0, sem1):
    w = lax.axis_index("s") * 2 + lax.axis_index("c")
    base = w * SPW

    pltpu.sync_copy(idx_hbm.at[pl.ds(base, SPW)], idx_v)

    rbufs = (rb0, rb1)
    sems = (sem0, sem1)

    def issue(c, b):
        pltpu.make_async_copy(
            emb_hbm.at[idx_v.at[c >> 2, c & 3]], rbufs[b], sems[b]).start()

    def wait(b):
        pltpu.make_async_copy(emb_hbm.at[idx_v.at[0, 0]], rbufs[b],
                              sems[b]).wait()

    def accum(b, init):
        rbuf = rbufs[b]

        def rbody(j, a):
            word = rbuf[j, :]
            lo = jax.lax.bitcast_convert_type(word << 16, jnp.float32)
            hi = jax.lax.bitcast_convert_type(
                word & jnp.int32(-65536), jnp.float32)
            return (a[0] + lo, a[1] + hi)

        return lax.fori_loop(0, HALF, rbody, init, unroll=5)

    issue(0, 0)
    issue(1, 1)

    zeros = (jnp.zeros((D // 2,), jnp.float32),
             jnp.zeros((D // 2,), jnp.float32))

    def outer(c2, _):
        i = c2 >> 1
        bag = c2 & 1

        c = 2 * c2
        wait(0)
        acc = accum(0, zeros)

        @pl.when(c + 2 < NCHUNK)
        def _():
            issue(c + 2, 0)

        wait(1)
        acc = accum(1, acc)

        @pl.when(c + 3 < NCHUNK)
        def _():
            issue(c + 3, 1)

        out_v[i, bag, pl.ds(0, D // 2)] = acc[0]
        out_v[i, bag, pl.ds(D // 2, D // 2)] = acc[1]
        return 0

    lax.fori_loop(0, NCHUNK // 2, outer, 0)

    pltpu.sync_copy(out_v, out_hbm.at[pl.ds(base, SPW)])


_bag_sum = functools.partial(
    pl.kernel,
    out_type=jax.ShapeDtypeStruct((B, 2, D), jnp.float32),
    mesh=plsc.VectorSubcoreMesh(core_axis_name="c", subcore_axis_name="s"),
    scratch_types=[
        pltpu.VMEM((SPW, 4, HALF), jnp.int32),
        pltpu.VMEM((HALF, D // 2), jnp.int32),
        pltpu.VMEM((HALF, D // 2), jnp.int32),
        pltpu.VMEM((SPW, 2, D), jnp.float32),
        pltpu.SemaphoreType.DMA,
        pltpu.SemaphoreType.DMA,
    ],
    compiler_params=pltpu.CompilerParams(use_tc_tiling_on_sc=False),
)(_bag_sum_body)


ROWS = 512


def _heads_body(pooled_ref, w1t_ref, b1_ref, ew1_ref, eb1_ref,
                ew2_ref, eb2_ref, ew3_ref, eb3_ref, y_ref, z_ref):
    h = jnp.maximum(pooled_ref[...], 0.0)
    hp = jax.lax.Precision.HIGHEST
    y_ref[...] = (jnp.dot(h, w1t_ref[...], precision=hp,
                          preferred_element_type=jnp.float32)
                  + b1_ref[...])
    z1 = jnp.maximum(jnp.dot(h, ew1_ref[...], precision=hp,
                             preferred_element_type=jnp.float32)
                     + eb1_ref[...], 0.0)
    z2 = jnp.maximum(jnp.dot(z1, ew2_ref[...], precision=hp,
                             preferred_element_type=jnp.float32)
                     + eb2_ref[...], 0.0)
    z_ref[...] = (jnp.dot(z2, ew3_ref[...], precision=hp,
                          preferred_element_type=jnp.float32)
                  + eb3_ref[...])


def _heads(pooled, w1t, b1, ew1, eb1, ew2, eb2, ew3, eb3):
    grid = (B // ROWS,)
    full = lambda shape: pl.BlockSpec(shape, lambda i: (0, 0))
    return pl.pallas_call(
        _heads_body,
        grid=grid,
        in_specs=[
            pl.BlockSpec((ROWS, 2 * D), lambda i: (i, 0)),
            full((2 * D, 4096)),
            full((1, 4096)),
            full((2 * D, D)),
            full((1, D)),
            full((D, D)),
            full((1, D)),
            full((D, D)),
            full((1, D)),
        ],
        out_specs=[
            pl.BlockSpec((ROWS, 4096), lambda i: (i, 0)),
            pl.BlockSpec((ROWS, D), lambda i: (i, 0)),
        ],
        out_shape=[
            jax.ShapeDtypeStruct((B, 4096), jnp.float32),
            jax.ShapeDtypeStruct((B, D), jnp.float32),
        ],
    )(pooled, w1t, b1, ew1, eb1, ew2, eb2, ew3, eb3)


_PERM_HALF = np.concatenate([np.arange(0, 128, 2), np.arange(1, 128, 2)])
_PERM = np.concatenate([_PERM_HALF, 128 + _PERM_HALF])


def kernel(x_w, x_b, emb, m_w1, m_b1, e_w1, e_b1, e_w2, e_b2, e_w3, e_b3):
    idx = jnp.concatenate(
        [x_w.reshape(B, 2, HALF), x_b.reshape(B, 2, HALF)], axis=1)
    u = jax.lax.bitcast_convert_type(emb, jnp.uint32)
    r = u + jnp.uint32(0x8000)
    w = (r[:, 0::2] >> 16) | (r[:, 1::2] & jnp.uint32(0xFFFF0000))
    tab = jax.lax.bitcast_convert_type(w, jnp.int32)
    pooled = _bag_sum(idx, tab).reshape(B, 2 * D)

    w1t = m_w1.T[_PERM]
    b1 = m_b1.reshape(1, 4096)
    ew1 = jnp.zeros((2 * D, D), jnp.float32).at[:, :32].set(e_w1.T)[_PERM]
    eb1 = jnp.zeros((1, D), jnp.float32).at[0, :32].set(e_b1)
    ew2 = jnp.zeros((D, D), jnp.float32).at[:32, :32].set(e_w2.T)
    eb2 = jnp.zeros((1, D), jnp.float32).at[0, :32].set(e_b2)
    ew3 = jnp.zeros((D, D), jnp.float32).at[:32, :1].set(e_w3.T)
    eb3 = jnp.zeros((1, D), jnp.float32).at[0, :1].set(e_b3)

    y, zfull = _heads(pooled, w1t, b1, ew1, eb1, ew2, eb2, ew3, eb3)
    return (y, zfull[:, :1])


# speedup vs baseline: 2.4747x; 2.4747x over previous
import functools

import jax
import jax.numpy as jnp
import numpy as np
from jax import lax
from jax.experimental import pallas as pl
from jax.experimental.pallas import tpu as pltpu
from jax.experimental.pallas import tpu_sc as plsc

B = 4096
L = 200
V = 40961
D = 128
HALF = 100
NW = 32
SPW = B // NW
NCHUNK = SPW * 4
LANES = 16


def _bag_sum_body(idx_hbm, emb_hbm, out_hbm,
                  idx_v, rb0, rb1, out_v, sem0, sem1):
    w = lax.axis_index("s") * 2 + lax.axis_index("c")
    base = w * SPW

    pltpu.sync_copy(idx_hbm.at[pl.ds(base, SPW)], idx_v)

    rbufs = (rb0, rb1)
    sems = (sem0, sem1)

    def issue(c, b):
        pltpu.make_async_copy(
            emb_hbm.at[idx_v.at[c >> 2, c & 3]], rbufs[b], sems[b]).start()

    def wait(b):
        pltpu.make_async_copy(emb_hbm.at[idx_v.at[0, 0]], rbufs[b],
                              sems[b]).wait()

    def accum(b, init):
        rbuf = rbufs[b]

        def rbody(j, a):
            word = rbuf[j, :]
            lo = jax.lax.bitcast_convert_type(word << 16, jnp.float32)
            hi = jax.lax.bitcast_convert_type(
                word & jnp.int32(-65536), jnp.float32)
            return (a[0] + lo, a[1] + hi)

        return lax.fori_loop(0, HALF, rbody, init, unroll=5)

    issue(0, 0)
    issue(1, 1)

    zeros = (jnp.zeros((D // 2,), jnp.float32),
             jnp.zeros((D // 2,), jnp.float32))

    def outer(c2, _):
        i = c2 >> 1
        bag = c2 & 1

        c = 2 * c2
        wait(0)
        acc = accum(0, zeros)

        @pl.when(c + 2 < NCHUNK)
        def _():
            issue(c + 2, 0)

        wait(1)
        acc = accum(1, acc)

        @pl.when(c + 3 < NCHUNK)
        def _():
            issue(c + 3, 1)

        out_v[i, bag, pl.ds(0, D // 2)] = acc[0]
        out_v[i, bag, pl.ds(D // 2, D // 2)] = acc[1]
        return 0

    lax.fori_loop(0, NCHUNK // 2, outer, 0)

    pltpu.sync_copy(out_v, out_hbm.at[pl.ds(base, SPW)])


_bag_sum = functools.partial(
    pl.kernel,
    out_type=jax.ShapeDtypeStruct((B, 2, D), jnp.float32),
    mesh=plsc.VectorSubcoreMesh(core_axis_name="c", subcore_axis_name="s"),
    scratch_types=[
        pltpu.VMEM((SPW, 4, HALF), jnp.int32),
        pltpu.VMEM((HALF, D // 2), jnp.int32),
        pltpu.VMEM((HALF, D // 2), jnp.int32),
        pltpu.VMEM((SPW, 2, D), jnp.float32),
        pltpu.SemaphoreType.DMA,
        pltpu.SemaphoreType.DMA,
    ],
    compiler_params=pltpu.CompilerParams(use_tc_tiling_on_sc=False),
)(_bag_sum_body)


ROWS = 512


def _heads_body(pooled_ref, w1t_ref, b1_ref, ew1_ref, eb1_ref,
                ew2_ref, eb2_ref, ew3_ref, eb3_ref, y_ref, z_ref):
    h = jnp.maximum(pooled_ref[...], 0.0)
    hp = jax.lax.Precision.HIGHEST
    y_ref[...] = (jnp.dot(h, w1t_ref[...], precision=hp,
                          preferred_element_type=jnp.float32)
                  + b1_ref[...])
    z1 = jnp.maximum(jnp.dot(h, ew1_ref[...], precision=hp,
                             preferred_element_type=jnp.float32)
                     + eb1_ref[...], 0.0)
    z2 = jnp.maximum(jnp.dot(z1, ew2_ref[...], precision=hp,
                             preferred_element_type=jnp.float32)
                     + eb2_ref[...], 0.0)
    z_ref[...] = (jnp.dot(z2, ew3_ref[...], precision=hp,
                          preferred_element_type=jnp.float32)
                  + eb3_ref[...])


def _heads(pooled, w1t, b1, ew1, eb1, ew2, eb2, ew3, eb3):
    grid = (B // ROWS,)
    full = lambda shape: pl.BlockSpec(shape, lambda i: (0, 0))
    return pl.pallas_call(
        _heads_body,
        grid=grid,
        in_specs=[
            pl.BlockSpec((ROWS, 2 * D), lambda i: (i, 0)),
            full((2 * D, 4096)),
            full((1, 4096)),
            full((2 * D, D)),
            full((1, D)),
            full((D, D)),
            full((1, D)),
            full((D, D)),
            full((1, D)),
        ],
        out_specs=[
            pl.BlockSpec((ROWS, 4096), lambda i: (i, 0)),
            pl.BlockSpec((ROWS, D), lambda i: (i, 0)),
        ],
        out_shape=[
            jax.ShapeDtypeStruct((B, 4096), jnp.float32),
            jax.ShapeDtypeStruct((B, D), jnp.float32),
        ],
    )(pooled, w1t, b1, ew1, eb1, ew2, eb2, ew3, eb3)


def kernel(x_w, x_b, emb, m_w1, m_b1, e_w1, e_b1, e_w2, e_b2, e_w3, e_b3):
    idx = jnp.concatenate(
        [x_w.reshape(B, 2, HALF), x_b.reshape(B, 2, HALF)], axis=1)
    r = jax.lax.bitcast_convert_type(emb, jnp.uint32) + jnp.uint32(0x8000)
    w = (r[:, :64] >> 16) | (r[:, 64:] & jnp.uint32(0xFFFF0000))
    tab = jax.lax.bitcast_convert_type(w, jnp.int32)
    pooled = _bag_sum(idx, tab).reshape(B, 2 * D)

    w1t = m_w1.T
    b1 = m_b1.reshape(1, 4096)
    ew1 = jnp.zeros((2 * D, D), jnp.float32).at[:, :32].set(e_w1.T)
    eb1 = jnp.zeros((1, D), jnp.float32).at[0, :32].set(e_b1)
    ew2 = jnp.zeros((D, D), jnp.float32).at[:32, :32].set(e_w2.T)
    eb2 = jnp.zeros((1, D), jnp.float32).at[0, :32].set(e_b2)
    ew3 = jnp.zeros((D, D), jnp.float32).at[:32, :1].set(e_w3.T)
    eb3 = jnp.zeros((1, D), jnp.float32).at[0, :1].set(e_b3)

    y, zfull = _heads(pooled, w1t, b1, ew1, eb1, ew2, eb2, ew3, eb3)
    return (y, zfull[:, :1])


# speedup vs baseline: 2.7294x; 1.1029x over previous
import functools

import jax
import jax.numpy as jnp
import numpy as np
from jax import lax
from jax.experimental import pallas as pl
from jax.experimental.pallas import tpu as pltpu
from jax.experimental.pallas import tpu_sc as plsc

B = 4096
L = 200
V = 40961
D = 128
HALF = 100
NW = 32
SPW = B // NW
NCHUNK = SPW * 4
LANES = 16


def _bag_sum_body(idx_hbm, emb_hbm, out_hbm,
                  idx_v, rb0, rb1, out_v, sem0, sem1):
    w = lax.axis_index("s") * 2 + lax.axis_index("c")
    base = w * SPW

    pltpu.sync_copy(idx_hbm.at[pl.ds(base, SPW)], idx_v)

    rbufs = (rb0, rb1)
    sems = (sem0, sem1)

    def issue(c, b):
        pltpu.make_async_copy(
            emb_hbm.at[idx_v.at[c >> 2, c & 3]], rbufs[b], sems[b]).start()

    def wait(b):
        pltpu.make_async_copy(emb_hbm.at[idx_v.at[0, 0]], rbufs[b],
                              sems[b]).wait()

    def accum(b, init):
        rbuf = rbufs[b]

        def rbody(j, a):
            word = rbuf[j, :]
            lo = jax.lax.bitcast_convert_type(word << 16, jnp.float32)
            hi = jax.lax.bitcast_convert_type(
                word & jnp.int32(-65536), jnp.float32)
            return (a[0] + lo, a[1] + hi)

        return lax.fori_loop(0, HALF, rbody, init, unroll=5)

    issue(0, 0)
    issue(1, 1)

    zeros = (jnp.zeros((D // 2,), jnp.float32),
             jnp.zeros((D // 2,), jnp.float32))

    def outer(c2, _):
        i = c2 >> 1
        bag = c2 & 1

        c = 2 * c2
        wait(0)
        acc = accum(0, zeros)

        @pl.when(c + 2 < NCHUNK)
        def _():
            issue(c + 2, 0)

        wait(1)
        acc = accum(1, acc)

        @pl.when(c + 3 < NCHUNK)
        def _():
            issue(c + 3, 1)

        out_v[i, bag, pl.ds(0, D // 2)] = acc[0]
        out_v[i, bag, pl.ds(D // 2, D // 2)] = acc[1]
        return 0

    lax.fori_loop(0, NCHUNK // 2, outer, 0)

    pltpu.sync_copy(out_v, out_hbm.at[pl.ds(base, SPW)])


_bag_sum = functools.partial(
    pl.kernel,
    out_type=jax.ShapeDtypeStruct((B, 2, D), jnp.float32),
    mesh=plsc.VectorSubcoreMesh(core_axis_name="c", subcore_axis_name="s"),
    scratch_types=[
        pltpu.VMEM((SPW, 4, HALF), jnp.int32),
        pltpu.VMEM((HALF, D // 2), jnp.int32),
        pltpu.VMEM((HALF, D // 2), jnp.int32),
        pltpu.VMEM((SPW, 2, D), jnp.float32),
        pltpu.SemaphoreType.DMA,
        pltpu.SemaphoreType.DMA,
    ],
    compiler_params=pltpu.CompilerParams(use_tc_tiling_on_sc=False),
)(_bag_sum_body)


ROWS = 512


def _heads_body(pooled_ref, w1t_ref, b1_ref, ew1_ref, eb1_ref,
                ew2_ref, eb2_ref, ew3_ref, eb3_ref, y_ref, z_ref):
    h = jnp.maximum(pooled_ref[...], 0.0)
    hp = jax.lax.Precision.DEFAULT
    y_ref[...] = (jnp.dot(h, w1t_ref[...], precision=hp,
                          preferred_element_type=jnp.float32)
                  + b1_ref[...])
    z1 = jnp.maximum(jnp.dot(h, ew1_ref[...], precision=hp,
                             preferred_element_type=jnp.float32)
                     + eb1_ref[...], 0.0)
    z2 = jnp.maximum(jnp.dot(z1, ew2_ref[...], precision=hp,
                             preferred_element_type=jnp.float32)
                     + eb2_ref[...], 0.0)
    z_ref[...] = (jnp.dot(z2, ew3_ref[...], precision=hp,
                          preferred_element_type=jnp.float32)
                  + eb3_ref[...])


def _heads(pooled, w1t, b1, ew1, eb1, ew2, eb2, ew3, eb3):
    grid = (B // ROWS,)
    full = lambda shape: pl.BlockSpec(shape, lambda i: (0, 0))
    return pl.pallas_call(
        _heads_body,
        grid=grid,
        in_specs=[
            pl.BlockSpec((ROWS, 2 * D), lambda i: (i, 0)),
            full((2 * D, 4096)),
            full((1, 4096)),
            full((2 * D, D)),
            full((1, D)),
            full((D, D)),
            full((1, D)),
            full((D, D)),
            full((1, D)),
        ],
        out_specs=[
            pl.BlockSpec((ROWS, 4096), lambda i: (i, 0)),
            pl.BlockSpec((ROWS, D), lambda i: (i, 0)),
        ],
        out_shape=[
            jax.ShapeDtypeStruct((B, 4096), jnp.float32),
            jax.ShapeDtypeStruct((B, D), jnp.float32),
        ],
    )(pooled, w1t, b1, ew1, eb1, ew2, eb2, ew3, eb3)


def kernel(x_w, x_b, emb, m_w1, m_b1, e_w1, e_b1, e_w2, e_b2, e_w3, e_b3):
    idx = jnp.concatenate(
        [x_w.reshape(B, 2, HALF), x_b.reshape(B, 2, HALF)], axis=1)
    r = jax.lax.bitcast_convert_type(emb, jnp.uint32) + jnp.uint32(0x8000)
    w = (r[:, :64] >> 16) | (r[:, 64:] & jnp.uint32(0xFFFF0000))
    tab = jax.lax.bitcast_convert_type(w, jnp.int32)
    pooled = _bag_sum(idx, tab).reshape(B, 2 * D)

    w1t = m_w1.T
    b1 = m_b1.reshape(1, 4096)
    ew1 = jnp.zeros((2 * D, D), jnp.float32).at[:, :32].set(e_w1.T)
    eb1 = jnp.zeros((1, D), jnp.float32).at[0, :32].set(e_b1)
    ew2 = jnp.zeros((D, D), jnp.float32).at[:32, :32].set(e_w2.T)
    eb2 = jnp.zeros((1, D), jnp.float32).at[0, :32].set(e_b2)
    ew3 = jnp.zeros((D, D), jnp.float32).at[:32, :1].set(e_w3.T)
    eb3 = jnp.zeros((1, D), jnp.float32).at[0, :1].set(e_b3)

    y, zfull = _heads(pooled, w1t, b1, ew1, eb1, ew2, eb2, ew3, eb3)
    return (y, zfull[:, :1])
